# Initial kernel scaffold; baseline (speedup 1.0000x reference)
import jax
import jax.numpy as jnp
from jax import lax
from jax.experimental import pallas as pl
from jax.experimental.pallas import tpu as pltpu
from jax.experimental.pallas import tpu_sc as plsc

N = 10000
E = 320000
D = 128
ROWW = 144
NC = 2
NS = 16
NW = NC * NS
CHUNK = 128
NCHUNK = -(-E // (NW * CHUNK))
EPT = NCHUNK * CHUNK
EPAD = EPT * NW
ACC_ROWS = 10240
TROWS = ACC_ROWS // NS
BR = 400


def _proj_block(h, att_row):
    a = jnp.sum(h * att_row, axis=1, keepdims=True) * 0.6
    return jnp.concatenate([h, jnp.broadcast_to(a, (h.shape[0], ROWW - D))],
                           axis=1)


def _proj_body(x_ref, wl_ref, wr_ref, att_ref, ol_ref, or_ref):
    xb = x_ref[...]
    att_row = att_ref[...]
    ol_ref[...] = _proj_block(
        jnp.dot(xb, wl_ref[...], preferred_element_type=jnp.float32), att_row)
    or_ref[...] = _proj_block(
        jnp.dot(xb, wr_ref[...], preferred_element_type=jnp.float32), att_row)


def _merge_block(a0, a1, b_row):
    p = a0 + a1
    num = p[:, :D]
    den = p[:, D:D + 1]
    return num / (den + 1e-16) + b_row


def _mid_body(a0_ref, a1_ref, b_ref, wl_ref, wr_ref, att_ref, ol_ref, or_ref):
    h = _merge_block(a0_ref[...], a1_ref[...], b_ref[...])
    h = jnp.where(h > 0, h, jnp.expm1(h))
    att_row = att_ref[...]
    ol_ref[...] = _proj_block(
        jnp.dot(h, wl_ref[...], preferred_element_type=jnp.float32), att_row)
    or_ref[...] = _proj_block(
        jnp.dot(h, wr_ref[...], preferred_element_type=jnp.float32), att_row)


def _fin_body(a0_ref, a1_ref, b_ref, o_ref):
    o_ref[...] = _merge_block(a0_ref[...], a1_ref[...], b_ref[...])


def _proj_call(x, Wl, Wr, att_row):
    return pl.pallas_call(
        _proj_body,
        grid=(N // BR,),
        in_specs=[
            pl.BlockSpec((BR, D), lambda i: (i, 0)),
            pl.BlockSpec((D, D), lambda i: (0, 0)),
            pl.BlockSpec((D, D), lambda i: (0, 0)),
            pl.BlockSpec((1, D), lambda i: (0, 0)),
        ],
        out_specs=[pl.BlockSpec((BR, ROWW), lambda i: (i, 0))] * 2,
        out_shape=[jax.ShapeDtypeStruct((N, ROWW), jnp.float32)] * 2,
    )(x, Wl, Wr, att_row)


def _mid_call(a0, a1, b_row, Wl, Wr, att_row):
    return pl.pallas_call(
        _mid_body,
        grid=(N // BR,),
        in_specs=[
            pl.BlockSpec((BR, ROWW), lambda i: (i, 0)),
            pl.BlockSpec((BR, ROWW), lambda i: (i, 0)),
            pl.BlockSpec((1, D), lambda i: (0, 0)),
            pl.BlockSpec((D, D), lambda i: (0, 0)),
            pl.BlockSpec((D, D), lambda i: (0, 0)),
            pl.BlockSpec((1, D), lambda i: (0, 0)),
        ],
        out_specs=[pl.BlockSpec((BR, ROWW), lambda i: (i, 0))] * 2,
        out_shape=[jax.ShapeDtypeStruct((N, ROWW), jnp.float32)] * 2,
    )(a0, a1, b_row, Wl, Wr, att_row)


def _fin_call(a0, a1, b_row):
    return pl.pallas_call(
        _fin_body,
        grid=(N // BR,),
        in_specs=[
            pl.BlockSpec((BR, ROWW), lambda i: (i, 0)),
            pl.BlockSpec((BR, ROWW), lambda i: (i, 0)),
            pl.BlockSpec((1, D), lambda i: (0, 0)),
        ],
        out_specs=pl.BlockSpec((BR, D), lambda i: (i, 0)),
        out_shape=jax.ShapeDtypeStruct((N, D), jnp.float32),
    )(a0, a1, b_row)


def _edge_body(xlp, xrp, srcr, dstr, attr, zr,
               out0, out1,
               sidx, didx, xj, xi, sc, lg, attv,
               acc,
               sem1, sem2):
    c = lax.axis_index("c")
    s = lax.axis_index("s")
    wid = c * NS + s

    pltpu.sync_copy(zr, acc.at[pl.ds(s * TROWS, TROWS)])
    pltpu.sync_copy(attr, attv)
    plsc.subcore_barrier()

    ebase = wid * EPT

    def chunk_body(k, carry):
        b = ebase + k * CHUNK
        pltpu.sync_copy(srcr.at[pl.ds(b, CHUNK)], sidx)
        pltpu.sync_copy(dstr.at[pl.ds(b, CHUNK)], didx)
        pltpu.async_copy(xlp.at[sidx], xj, sem1).wait()
        pltpu.async_copy(xrp.at[didx], xi, sem2).wait()

        def edge_logit(e, cc):
            a16 = jnp.zeros((16,), jnp.float32)
            for j in range(D // 16):
                z = xj[e, pl.ds(16 * j, 16)] + xi[e, pl.ds(16 * j, 16)]
                a16 = a16 + attv[pl.ds(16 * j, 16)] * jnp.abs(z)
            lg[e] = xj[e, D] + xi[e, D] + jnp.sum(a16)
            return cc

        lax.fori_loop(0, CHUNK, edge_logit, 0)

        for g in range(CHUNK // 16):
            lg[pl.ds(16 * g, 16)] = jnp.exp(lg[pl.ds(16 * g, 16)])

        def edge_scale(e, cc):
            ex = lg[e]
            for j in range(D // 16):
                sc[e, pl.ds(16 * j, 16)] = xj[e, pl.ds(16 * j, 16)] * ex
            sc[e, D] = ex
            return cc

        lax.fori_loop(0, CHUNK, edge_scale, 0)

        pltpu.sync_copy(sc, acc.at[didx], add=True)
        return carry

    lax.fori_loop(0, NCHUNK, chunk_body, 0)
    plsc.subcore_barrier()

    @pl.when(c == 0)
    def _():
        pltpu.sync_copy(acc.at[pl.ds(s * TROWS, TROWS)],
                        out0.at[pl.ds(s * TROWS, TROWS)])

    @pl.when(c == 1)
    def _():
        pltpu.sync_copy(acc.at[pl.ds(s * TROWS, TROWS)],
                        out1.at[pl.ds(s * TROWS, TROWS)])


def _edge_call(xlp, xrp, src, dst, attv, zrows):
    mesh = plsc.VectorSubcoreMesh(core_axis_name="c", subcore_axis_name="s")
    return pl.kernel(
        _edge_body,
        out_type=[jax.ShapeDtypeStruct((ACC_ROWS, ROWW), jnp.float32)] * 2,
        mesh=mesh,
        scratch_types=[
            pltpu.VMEM((CHUNK,), jnp.int32),
            pltpu.VMEM((CHUNK,), jnp.int32),
            pltpu.VMEM((CHUNK, ROWW), jnp.float32),
            pltpu.VMEM((CHUNK, ROWW), jnp.float32),
            pltpu.VMEM((CHUNK, ROWW), jnp.float32),
            pltpu.VMEM((CHUNK,), jnp.float32),
            pltpu.VMEM((D,), jnp.float32),
            pltpu.VMEM_SHARED((ACC_ROWS, ROWW), jnp.float32),
            pltpu.SemaphoreType.DMA,
            pltpu.SemaphoreType.DMA,
        ],
    )(xlp, xrp, src, dst, attv, zrows)


def kernel(x, edge_index, Wl1, Wr1, att1, b1, Wl2, Wr2, att2, b2):
    x = x.reshape(-1, x.shape[-1])
    ei = edge_index.astype(jnp.int32)
    src = ei[:, 0]
    dst = ei[:, 1]
    npad = EPAD - E
    src = jnp.concatenate([src, jnp.zeros((npad,), jnp.int32)])
    dst = jnp.concatenate([dst, jnp.full((npad,), N, jnp.int32)])
    zrows = jnp.zeros((TROWS, ROWW), jnp.float32)

    att1_row = att1.reshape(1, D)
    att2_row = att2.reshape(1, D)
    attv1 = att1_row.reshape(D) * 0.4
    attv2 = att2_row.reshape(D) * 0.4

    xlp1, xrp1 = _proj_call(x, Wl1, Wr1, att1_row)
    accA1, accB1 = _edge_call(xlp1, xrp1, src, dst, attv1, zrows)
    xlp2, xrp2 = _mid_call(accA1, accB1, b1.reshape(1, D), Wl2, Wr2, att2_row)
    accA2, accB2 = _edge_call(xlp2, xrp2, src, dst, attv2, zrows)
    return _fin_call(accA2, accB2, b2.reshape(1, D))


# speedup vs baseline: 7.2249x; 7.2249x over previous
import jax
import jax.numpy as jnp
from jax import lax
from jax.experimental import pallas as pl
from jax.experimental.pallas import tpu as pltpu
from jax.experimental.pallas import tpu_sc as plsc

N = 10000
E = 320000
D = 128
NC = 2
NS = 16
NW = NC * NS
CHUNK = 128
NCHUNK = -(-E // (NW * CHUNK))
EPT = NCHUNK * CHUNK
EPAD = EPT * NW
DEN_BASE = 10112
DROWS = 80
ACC_ROWS = 10240
TROWS = ACC_ROWS // NS
BR = 1024
GRID = -(-N // BR)
SUB = 128
DUMMY_DST = N


def _row_to_col(r):
    b = r.shape[1]
    ri = lax.broadcasted_iota(jnp.int32, (b, b), 0)
    ci = lax.broadcasted_iota(jnp.int32, (b, b), 1)
    m = jnp.where(ri == ci, jnp.broadcast_to(r, (b, b)), 0.0)
    return jnp.sum(m, axis=1, keepdims=True)


def _merge_block(n0_ref, d0_ref, n1_ref, d1_ref, b_ref):
    num = n0_ref[...] + n1_ref[...]
    dsum = d0_ref[...] + d1_ref[...]
    subs = []
    for r in range(BR // SUB):
        den = _row_to_col(dsum[r:r + 1, :])
        subs.append(num[r * SUB:(r + 1) * SUB, :] / (den + 1e-16))
    return jnp.concatenate(subs, axis=0) + b_ref[...]


def _proj_body(x_ref, wl_ref, wr_ref, ol_ref, or_ref):
    xb = x_ref[...]
    ol_ref[...] = jnp.dot(xb, wl_ref[...], preferred_element_type=jnp.float32)
    or_ref[...] = jnp.dot(xb, wr_ref[...], preferred_element_type=jnp.float32)


def _mid_body(n0_ref, d0_ref, n1_ref, d1_ref, b_ref, wl_ref, wr_ref,
              ol_ref, or_ref):
    h = _merge_block(n0_ref, d0_ref, n1_ref, d1_ref, b_ref)
    h = jnp.where(h > 0, h, jnp.exp(h) - 1.0)
    ol_ref[...] = jnp.dot(h, wl_ref[...], preferred_element_type=jnp.float32)
    or_ref[...] = jnp.dot(h, wr_ref[...], preferred_element_type=jnp.float32)


def _fin_body(n0_ref, d0_ref, n1_ref, d1_ref, b_ref, o_ref):
    o_ref[...] = _merge_block(n0_ref, d0_ref, n1_ref, d1_ref, b_ref)


_W_SPEC = pl.BlockSpec((D, D), lambda i: (0, 0))
_V_SPEC = pl.BlockSpec((1, D), lambda i: (0, 0))
_X_SPEC = pl.BlockSpec((BR, D), lambda i: (i, 0))
_NUM_SPEC = pl.BlockSpec((BR, D), lambda i: (i, 0))
_DEN_SPEC = pl.BlockSpec((BR // SUB, D), lambda i: (DEN_BASE // (BR // SUB) + i, 0))

_PROJ_OUT = [jax.ShapeDtypeStruct((N, D), jnp.float32)] * 2
_PROJ_OUT_SPECS = [_X_SPEC, _X_SPEC]


def _proj_call(x, Wl, Wr):
    return pl.pallas_call(
        _proj_body,
        grid=(GRID,),
        in_specs=[_X_SPEC, _W_SPEC, _W_SPEC],
        out_specs=_PROJ_OUT_SPECS,
        out_shape=_PROJ_OUT,
    )(x, Wl, Wr)


def _mid_call(acc0, acc1, b_row, Wl, Wr):
    return pl.pallas_call(
        _mid_body,
        grid=(GRID,),
        in_specs=[_NUM_SPEC, _DEN_SPEC, _NUM_SPEC, _DEN_SPEC,
                  _V_SPEC, _W_SPEC, _W_SPEC],
        out_specs=_PROJ_OUT_SPECS,
        out_shape=_PROJ_OUT,
    )(acc0, acc0, acc1, acc1, b_row, Wl, Wr)


def _fin_call(acc0, acc1, b_row):
    return pl.pallas_call(
        _fin_body,
        grid=(GRID,),
        in_specs=[_NUM_SPEC, _DEN_SPEC, _NUM_SPEC, _DEN_SPEC, _V_SPEC],
        out_specs=pl.BlockSpec((BR, D), lambda i: (i, 0)),
        out_shape=jax.ShapeDtypeStruct((N, D), jnp.float32),
    )(acc0, acc0, acc1, acc1, b_row)


def _edge_body(xlp, xrp, srcr, dstr, attr, zr,
               out0, out1,
               sidx, didx, xj, xi, lg, attv, denv, dmidx,
               acc,
               sem1, sem2):
    c = lax.axis_index("c")
    s = lax.axis_index("s")
    wid = c * NS + s
    lane = lax.iota(jnp.int32, 16)
    perms = [(lane + sh) & 15 for sh in (8, 4, 2, 1)]
    gd = lax.GatherDimensionNumbers(offset_dims=(), collapsed_slice_dims=(0,),
                                    start_index_map=(0,))

    def lanesum(v):
        for p in perms:
            v = v + lax.gather(v, p[:, None], gd, slice_sizes=(1,),
                               mode=lax.GatherScatterMode.PROMISE_IN_BOUNDS)
        return v[0]

    pltpu.sync_copy(zr.at[pl.ds(0, TROWS)], acc.at[pl.ds(s * TROWS, TROWS)])
    pltpu.sync_copy(attr, attv)
    pltpu.sync_copy(zr.at[pl.ds(0, DROWS)], denv)
    for q in range(DROWS // 16):
        dmidx[pl.ds(16 * q, 16)] = DEN_BASE + 16 * q + lane
    plsc.subcore_barrier()

    ebase = wid * EPT

    def chunk_body(k, carry):
        b = ebase + k * CHUNK
        pltpu.sync_copy(srcr.at[pl.ds(b, CHUNK)], sidx)
        pltpu.sync_copy(dstr.at[pl.ds(b, CHUNK)], didx)
        pltpu.async_copy(xlp.at[sidx], xj, sem1).wait()
        pltpu.async_copy(xrp.at[didx], xi, sem2).wait()

        def group_logit(g, cc):
            dv = didx[pl.ds(g * 16, 16)]
            lgvec = jnp.zeros((16,), jnp.float32)
            for t in range(16):
                e = g * 16 + t
                a16 = jnp.zeros((16,), jnp.float32)
                for j in range(D // 16):
                    z = xj[e, pl.ds(16 * j, 16)] + xi[e, pl.ds(16 * j, 16)]
                    lr = jnp.maximum(z, 0.0) + 0.2 * jnp.minimum(z, 0.0)
                    a16 = a16 + attv[pl.ds(16 * j, 16)] * lr
                lgvec = jnp.where(lane == t, lanesum(a16), lgvec)
            ev = jnp.exp(lgvec)
            lg[pl.ds(g * 16, 16)] = ev
            rr = lax.shift_right_logical(dv, 7)
            cc16 = dv & 127
            for t in range(16):
                plsc.addupdate_scatter(denv, [rr, cc16], ev, mask=lane == t)
            return cc

        lax.fori_loop(0, CHUNK // 16, group_logit, 0)

        def group_scale(g, cc):
            lgv = lg[pl.ds(g * 16, 16)]
            for t in range(16):
                e = g * 16 + t
                ex = lgv[t]
                for j in range(D // 16):
                    xj[e, pl.ds(16 * j, 16)] = xj[e, pl.ds(16 * j, 16)] * ex
            return cc

        lax.fori_loop(0, CHUNK // 16, group_scale, 0)

        pltpu.sync_copy(xj, acc.at[didx], add=True)
        return carry

    lax.fori_loop(0, NCHUNK, chunk_body, 0)

    pltpu.sync_copy(denv, acc.at[dmidx], add=True)
    plsc.subcore_barrier()

    @pl.when(c == 0)
    def _():
        pltpu.sync_copy(acc.at[pl.ds(s * TROWS, TROWS)],
                        out0.at[pl.ds(s * TROWS, TROWS)])

    @pl.when(c == 1)
    def _():
        pltpu.sync_copy(acc.at[pl.ds(s * TROWS, TROWS)],
                        out1.at[pl.ds(s * TROWS, TROWS)])


def _edge_call(xlp, xrp, src, dst, attv, zrows):
    mesh = plsc.VectorSubcoreMesh(core_axis_name="c", subcore_axis_name="s")
    return pl.kernel(
        _edge_body,
        out_type=[jax.ShapeDtypeStruct((ACC_ROWS, D), jnp.float32)] * 2,
        mesh=mesh,
        compiler_params=pltpu.CompilerParams(needs_layout_passes=False),
        scratch_types=[
            pltpu.VMEM((CHUNK,), jnp.int32),
            pltpu.VMEM((CHUNK,), jnp.int32),
            pltpu.VMEM((CHUNK, D), jnp.float32),
            pltpu.VMEM((CHUNK, D), jnp.float32),
            pltpu.VMEM((CHUNK,), jnp.float32),
            pltpu.VMEM((D,), jnp.float32),
            pltpu.VMEM((DROWS, D), jnp.float32),
            pltpu.VMEM((DROWS,), jnp.int32),
            pltpu.VMEM_SHARED((ACC_ROWS, D), jnp.float32),
            pltpu.SemaphoreType.DMA,
            pltpu.SemaphoreType.DMA,
        ],
    )(xlp, xrp, src, dst, attv, zrows)


def kernel(x, edge_index, Wl1, Wr1, att1, b1, Wl2, Wr2, att2, b2):
    x = x.reshape(-1, x.shape[-1])
    ei = edge_index.astype(jnp.int32)
    src = ei[:, 0]
    dst = ei[:, 1]
    npad = EPAD - E
    src = jnp.concatenate([src, jnp.zeros((npad,), jnp.int32)])
    dst = jnp.concatenate([dst, jnp.full((npad,), DUMMY_DST, jnp.int32)])
    zrows = jnp.zeros((TROWS, D), jnp.float32)

    attv1 = att1.reshape(D)
    attv2 = att2.reshape(D)

    xlp1, xrp1 = _proj_call(x, Wl1, Wr1)
    acc0, acc1 = _edge_call(xlp1, xrp1, src, dst, attv1, zrows)
    xlp2, xrp2 = _mid_call(acc0, acc1, b1.reshape(1, D), Wl2, Wr2)
    acc0b, acc1b = _edge_call(xlp2, xrp2, src, dst, attv2, zrows)
    return _fin_call(acc0b, acc1b, b2.reshape(1, D))


# speedup vs baseline: 12.9151x; 1.7876x over previous
import jax
import jax.numpy as jnp
from jax import lax
from jax.experimental import pallas as pl
from jax.experimental.pallas import tpu as pltpu
from jax.experimental.pallas import tpu_sc as plsc

N = 10000
E = 320000
D = 128
NC = 2
NS = 16
NW = NC * NS
CHUNK = 64
CPB = 8
NCHUNK = 160
EPT = NCHUNK * CHUNK
EPAD = EPT * NW
DEN_BASE = 10112
DROWS = 80
ACC_ROWS = 10240
TROWS = ACC_ROWS // NS
BR = 1024
GRID = -(-N // BR)
SUB = 128
DUMMY_DST = N


def _row_to_col(r):
    b = r.shape[1]
    ri = lax.broadcasted_iota(jnp.int32, (b, b), 0)
    ci = lax.broadcasted_iota(jnp.int32, (b, b), 1)
    m = jnp.where(ri == ci, jnp.broadcast_to(r, (b, b)), 0.0)
    return jnp.sum(m, axis=1, keepdims=True)


def _merge_block(n0_ref, d0_ref, n1_ref, d1_ref, b_ref):
    num = n0_ref[...] + n1_ref[...]
    dsum = d0_ref[...] + d1_ref[...]
    subs = []
    for r in range(BR // SUB):
        den = _row_to_col(dsum[r:r + 1, :])
        subs.append(num[r * SUB:(r + 1) * SUB, :] / (den + 1e-16))
    return jnp.concatenate(subs, axis=0) + b_ref[...]


def _proj_body(x_ref, wl_ref, wr_ref, ol_ref, or_ref):
    xb = x_ref[...]
    ol_ref[...] = jnp.dot(xb, wl_ref[...], preferred_element_type=jnp.float32)
    or_ref[...] = jnp.dot(xb, wr_ref[...], preferred_element_type=jnp.float32)


def _mid_body(n0_ref, d0_ref, n1_ref, d1_ref, b_ref, wl_ref, wr_ref,
              ol_ref, or_ref):
    h = _merge_block(n0_ref, d0_ref, n1_ref, d1_ref, b_ref)
    h = jnp.where(h > 0, h, jnp.exp(h) - 1.0)
    ol_ref[...] = jnp.dot(h, wl_ref[...], preferred_element_type=jnp.float32)
    or_ref[...] = jnp.dot(h, wr_ref[...], preferred_element_type=jnp.float32)


def _fin_body(n0_ref, d0_ref, n1_ref, d1_ref, b_ref, o_ref):
    o_ref[...] = _merge_block(n0_ref, d0_ref, n1_ref, d1_ref, b_ref)


_W_SPEC = pl.BlockSpec((D, D), lambda i: (0, 0))
_V_SPEC = pl.BlockSpec((1, D), lambda i: (0, 0))
_X_SPEC = pl.BlockSpec((BR, D), lambda i: (i, 0))
_NUM_SPEC = pl.BlockSpec((BR, D), lambda i: (i, 0))
_DEN_SPEC = pl.BlockSpec((BR // SUB, D), lambda i: (DEN_BASE // (BR // SUB) + i, 0))

_PROJ_OUT = [jax.ShapeDtypeStruct((N, D), jnp.float32)] * 2
_PROJ_OUT_SPECS = [_X_SPEC, _X_SPEC]


def _proj_call(x, Wl, Wr):
    return pl.pallas_call(
        _proj_body,
        grid=(GRID,),
        in_specs=[_X_SPEC, _W_SPEC, _W_SPEC],
        out_specs=_PROJ_OUT_SPECS,
        out_shape=_PROJ_OUT,
    )(x, Wl, Wr)


def _mid_call(acc0, acc1, b_row, Wl, Wr):
    return pl.pallas_call(
        _mid_body,
        grid=(GRID,),
        in_specs=[_NUM_SPEC, _DEN_SPEC, _NUM_SPEC, _DEN_SPEC,
                  _V_SPEC, _W_SPEC, _W_SPEC],
        out_specs=_PROJ_OUT_SPECS,
        out_shape=_PROJ_OUT,
    )(acc0, acc0, acc1, acc1, b_row, Wl, Wr)


def _fin_call(acc0, acc1, b_row):
    return pl.pallas_call(
        _fin_body,
        grid=(GRID,),
        in_specs=[_NUM_SPEC, _DEN_SPEC, _NUM_SPEC, _DEN_SPEC, _V_SPEC],
        out_specs=pl.BlockSpec((BR, D), lambda i: (i, 0)),
        out_shape=jax.ShapeDtypeStruct((N, D), jnp.float32),
    )(acc0, acc0, acc1, acc1, b_row)


def _edge_body(xlp, xrp, srcr, dstr, attr, zr,
               out0, out1,
               sidxb, didxb, xj0, xj1, xi0, xi1,
               attv, denv, dmidx,
               acc,
               saj, sai, sbj, sbi, sxs, sxd):
    c = lax.axis_index("c")
    s = lax.axis_index("s")
    wid = c * NS + s
    lane = lax.iota(jnp.int32, 16)
    perms = [(lane + sh) & 15 for sh in (8, 4, 2, 1)]
    gd = lax.GatherDimensionNumbers(offset_dims=(), collapsed_slice_dims=(0,),
                                    start_index_map=(0,))

    def lanesum(v):
        for p in perms:
            v = v + lax.gather(v, p[:, None], gd, slice_sizes=(1,),
                               mode=lax.GatherScatterMode.PROMISE_IN_BOUNDS)
        return v[0]

    pltpu.sync_copy(zr.at[pl.ds(0, TROWS)], acc.at[pl.ds(s * TROWS, TROWS)])
    pltpu.sync_copy(attr, attv)
    z16 = jnp.zeros((16,), jnp.float32)

    def zero_den(r, cc):
        for j in range(D // 16):
            denv[r, pl.ds(16 * j, 16)] = z16
        return cc

    lax.fori_loop(0, DROWS, zero_den, 0)
    for q in range(5):
        dmidx[pl.ds(16 * q, 16)] = DEN_BASE + 16 * q + lane
    plsc.subcore_barrier()

    rbase = wid * NCHUNK
    NBLK = NCHUNK // CPB
    realk = NCHUNK

    def load_block_async(bid):
        q = bid & 1
        pltpu.async_copy(srcr.at[pl.ds(rbase + bid * CPB, CPB)],
                         sidxb.at[q], sxs)
        pltpu.async_copy(dstr.at[pl.ds(rbase + bid * CPB, CPB)],
                         didxb.at[q], sxd)

    def wait_block(bid):
        q = bid & 1
        pltpu.make_async_copy(srcr.at[pl.ds(rbase + bid * CPB, CPB)],
                              sidxb.at[q], sxs).wait()
        pltpu.make_async_copy(dstr.at[pl.ds(rbase + bid * CPB, CPB)],
                              didxb.at[q], sxd).wait()

    def srow(k):
        return sidxb.at[(k // CPB) & 1, k % CPB]

    def drow(k):
        return didxb.at[(k // CPB) & 1, k % CPB]

    def issue(k, xjb, xib, semj, semi):
        pltpu.async_copy(xlp.at[srow(k)], xjb, semj)
        pltpu.async_copy(xrp.at[drow(k)], xib, semi)

    def process(k, xjb, xib, semj, semi):
        pltpu.make_async_copy(xlp.at[srow(k)], xjb, semj).wait()
        pltpu.make_async_copy(xrp.at[drow(k)], xib, semi).wait()

        def group_body(g, cc):
            dv = didxb[(k // CPB) & 1, k % CPB, pl.ds(g * 16, 16)]
            rr = lax.shift_right_logical(dv, 7)
            cc16 = dv & 127
            for t in range(16):
                e = g * 16 + t
                xjc = [xjb[e, pl.ds(16 * j, 16)] for j in range(D // 16)]
                a16 = jnp.zeros((16,), jnp.float32)
                for j in range(D // 16):
                    z = xjc[j] + xib[e, pl.ds(16 * j, 16)]
                    lr = jnp.where(z > 0, z, z * 0.2)
                    a16 = a16 + attv[pl.ds(16 * j, 16)] * lr
                ex16 = jnp.exp(jnp.broadcast_to(lanesum(a16), (16,)))
                plsc.addupdate_scatter(denv, [rr, cc16], ex16, mask=lane == t)
                for j in range(D // 16):
                    xjb[e, pl.ds(16 * j, 16)] = xjc[j] * ex16
            return cc

        lax.fori_loop(0, CHUNK // 16, group_body, 0)
        pltpu.sync_copy(xjb, acc.at[drow(k)], add=True)

    load_block_async(0)
    wait_block(0)
    load_block_async(1)
    issue(0, xj0, xi0, saj, sai)
    issue(1, xj1, xi1, sbj, sbi)

    def pair_body(p, carry):
        k0 = 2 * p

        @pl.when(k0 < realk)
        def _():
            process(k0, xj0, xi0, saj, sai)

        @pl.when(p < NCHUNK // 2 - 1)
        def _():
            @pl.when((k0 + 2) % CPB == 0)
            def _():
                wait_block((k0 + 2) // CPB)

            @pl.when(k0 + 2 < realk)
            def _():
                issue(k0 + 2, xj0, xi0, saj, sai)

        @pl.when(k0 + 1 < realk)
        def _():
            process(k0 + 1, xj1, xi1, sbj, sbi)

        @pl.when(p < NCHUNK // 2 - 1)
        def _():
            @pl.when(k0 + 3 < realk)
            def _():
                issue(k0 + 3, xj1, xi1, sbj, sbi)

            @pl.when((k0 + 2) % CPB == 0)
            def _():
                nxt = (k0 + 2) // CPB + 1

                @pl.when(nxt < NBLK)
                def _():
                    load_block_async(nxt)

        return carry

    lax.fori_loop(0, NCHUNK // 2, pair_body, 0)

    pltpu.sync_copy(denv, acc.at[dmidx], add=True)
    plsc.subcore_barrier()

    @pl.when(c == 0)
    def _():
        pltpu.sync_copy(acc.at[pl.ds(s * TROWS, TROWS)],
                        out0.at[pl.ds(s * TROWS, TROWS)])

    @pl.when(c == 1)
    def _():
        pltpu.sync_copy(acc.at[pl.ds(s * TROWS, TROWS)],
                        out1.at[pl.ds(s * TROWS, TROWS)])


def _edge_call(xlp, xrp, src2d, dst2d, attv, zrows):
    mesh = plsc.VectorSubcoreMesh(core_axis_name="c", subcore_axis_name="s")
    return pl.kernel(
        _edge_body,
        out_type=[jax.ShapeDtypeStruct((ACC_ROWS, D), jnp.float32)] * 2,
        mesh=mesh,
        compiler_params=pltpu.CompilerParams(needs_layout_passes=False),
        scratch_types=[
            pltpu.VMEM((2, CPB, CHUNK), jnp.int32),
            pltpu.VMEM((2, CPB, CHUNK), jnp.int32),
            pltpu.VMEM((CHUNK, D), jnp.float32),
            pltpu.VMEM((CHUNK, D), jnp.float32),
            pltpu.VMEM((CHUNK, D), jnp.float32),
            pltpu.VMEM((CHUNK, D), jnp.float32),
            pltpu.VMEM((D,), jnp.float32),
            pltpu.VMEM((DROWS, D), jnp.float32),
            pltpu.VMEM((DROWS,), jnp.int32),
            pltpu.VMEM_SHARED((ACC_ROWS, D), jnp.float32),
            pltpu.SemaphoreType.DMA,
            pltpu.SemaphoreType.DMA,
            pltpu.SemaphoreType.DMA,
            pltpu.SemaphoreType.DMA,
            pltpu.SemaphoreType.DMA,
            pltpu.SemaphoreType.DMA,
        ],
    )(xlp, xrp, src2d, dst2d, attv, zrows)


def kernel(x, edge_index, Wl1, Wr1, att1, b1, Wl2, Wr2, att2, b2):
    x = x.reshape(-1, x.shape[-1])
    ei = edge_index.astype(jnp.int32)
    src = ei[:, 0]
    dst = ei[:, 1]
    npad = EPAD - E
    pad_i = jnp.arange(npad, dtype=jnp.int32) & 7
    src = jnp.concatenate([src, pad_i])
    dst = jnp.concatenate([dst, DUMMY_DST + pad_i])
    src = src.reshape(EPAD // CHUNK, CHUNK)
    dst = dst.reshape(EPAD // CHUNK, CHUNK)
    zrows = jnp.zeros((TROWS, D), jnp.float32)

    attv1 = att1.reshape(D)
    attv2 = att2.reshape(D)

    xlp1, xrp1 = _proj_call(x, Wl1, Wr1)
    acc0, acc1 = _edge_call(xlp1, xrp1, src, dst, attv1, zrows)
    xlp2, xrp2 = _mid_call(acc0, acc1, b1.reshape(1, D), Wl2, Wr2)
    acc0b, acc1b = _edge_call(xlp2, xrp2, src, dst, attv2, zrows)
    return _fin_call(acc0b, acc1b, b2.reshape(1, D))
